# Initial kernel scaffold; baseline (speedup 1.0000x reference)
import jax
import jax.numpy as jnp
from jax.experimental import pallas as pl


def kernel(x, x_mask, edge_index, edge_attr, batch, W_enc, b_enc, W_self, W_nei, b_mp, W_dec, b_dec):
    raise NotImplementedError("write your pallas kernel here")


# speedup vs baseline: 4.4263x; 4.4263x over previous
import functools

import jax
import jax.numpy as jnp
from jax import lax
from jax.experimental import pallas as pl
from jax.experimental.pallas import tpu as pltpu
from jax.experimental.pallas import tpu_sc as plsc

_N = 10000
_E = 320000
_DH = 128
_NW = 32
_C = 128
_KC = 79
_EPW = _KC * _C
_EPAD = _NW * _EPW
_NPAD = 10240
_RPT = _NPAD // 16
_ROWBLK = 1000
_GRID = _N // _ROWBLK


def _pre_body(x1, x2, w1, w2, be, wsn, bm, hw_ref, hn_ref):
    h = jnp.maximum(
        jnp.dot(x1[...], w1[...], preferred_element_type=jnp.float32)
        + jnp.dot(x2[...], w2[...], preferred_element_type=jnp.float32)
        + be[...], 0.0)
    hsn = jnp.dot(h, wsn[...], preferred_element_type=jnp.float32)
    hw_ref[...] = hsn[:, :_DH] + bm[...]
    hn_ref[...] = hsn[:, _DH:]


def _mid_body(hw, p0, p1, wsn, bm, hw_ref, hn_ref):
    h = jnp.maximum(hw[...] + p0[0] + p1[0], 0.0)
    hsn = jnp.dot(h, wsn[...], preferred_element_type=jnp.float32)
    hw_ref[...] = hsn[:, :_DH] + bm[...]
    hn_ref[...] = hsn[:, _DH:]


def _fin_body(hw, p0, p1, wd, bd, o_ref):
    h = jnp.maximum(hw[...] + p0[0] + p1[0], 0.0)
    o_ref[...] = jnp.dot(h, wd[...], preferred_element_type=jnp.float32) + bd[...]


_row_spec = pl.BlockSpec((_ROWBLK, _DH), lambda i: (i, 0))
_w_spec = pl.BlockSpec((_DH, _DH), lambda i: (0, 0))
_wsn_spec = pl.BlockSpec((_DH, 2 * _DH), lambda i: (0, 0))
_b_spec = pl.BlockSpec((1, _DH), lambda i: (0, 0))
_p0_spec = pl.BlockSpec((1, _ROWBLK, _DH), lambda i: (0, i, 0))
_p1_spec = pl.BlockSpec((1, _ROWBLK, _DH), lambda i: (1, i, 0))

_hh_out = (jax.ShapeDtypeStruct((_N, _DH), jnp.float32),
           jax.ShapeDtypeStruct((_N, _DH), jnp.float32))


def _tc_pre(x1, x2, w1, w2, be, wsn, bm):
    return pl.pallas_call(
        _pre_body,
        grid=(_GRID,),
        in_specs=[_row_spec, _row_spec, _w_spec, _w_spec, _b_spec, _wsn_spec, _b_spec],
        out_specs=(_row_spec, _row_spec),
        out_shape=_hh_out,
    )(x1, x2, w1, w2, be, wsn, bm)


def _tc_mid(hw, parts, wsn, bm):
    return pl.pallas_call(
        _mid_body,
        grid=(_GRID,),
        in_specs=[_row_spec, _p0_spec, _p1_spec, _wsn_spec, _b_spec],
        out_specs=(_row_spec, _row_spec),
        out_shape=_hh_out,
    )(hw, parts, parts, wsn, bm)


def _tc_fin(hw, parts, wd, bd):
    return pl.pallas_call(
        _fin_body,
        grid=(_GRID,),
        in_specs=[_row_spec, _p0_spec, _p1_spec,
                  pl.BlockSpec((_DH, 8), lambda i: (0, 0)),
                  pl.BlockSpec((1, 8), lambda i: (0, 0))],
        out_specs=pl.BlockSpec((_ROWBLK, 8), lambda i: (i, 0)),
        out_shape=jax.ShapeDtypeStruct((_N, 8), jnp.float32),
    )(hw, parts, parts, wd, bd)


def _sc_body(hn_hbm, src_hbm, dst_hbm, z_hbm, out_hbm, src_v, dst_v, rows_v, agg, gsem):
    cid = lax.axis_index("c")
    sid = lax.axis_index("s")
    wid = sid * 2 + cid
    base = sid * _RPT
    pltpu.sync_copy(z_hbm.at[pl.ds(base, _RPT)], agg.at[pl.ds(base, _RPT)])
    pltpu.sync_copy(src_hbm.at[wid], src_v)
    pltpu.sync_copy(dst_hbm.at[wid], dst_v)
    plsc.subcore_barrier()

    def chunk(j, carry):
        pltpu.async_copy(hn_hbm.at[src_v.at[j]], rows_v, gsem).wait()
        pltpu.sync_copy(rows_v, agg.at[dst_v.at[j]], add=True)
        return carry

    lax.fori_loop(0, _KC, chunk, 0)
    plsc.subcore_barrier()
    pltpu.sync_copy(agg.at[pl.ds(base, _RPT)],
                    out_hbm.at[cid, pl.ds(base, _RPT)])


def _sc_segsum(hn, src3, dst3, zeros):
    kern = pl.kernel(
        _sc_body,
        out_type=jax.ShapeDtypeStruct((2, _NPAD, _DH), jnp.float32),
        mesh=plsc.VectorSubcoreMesh(core_axis_name="c", subcore_axis_name="s"),
        scratch_types=[
            pltpu.VMEM((_KC, _C), jnp.int32),
            pltpu.VMEM((_KC, _C), jnp.int32),
            pltpu.VMEM((_C, _DH), jnp.float32),
            pltpu.VMEM_SHARED((_NPAD, _DH), jnp.float32),
            pltpu.SemaphoreType.DMA,
        ],
    )
    return kern(hn, src3, dst3, zeros)


def kernel(x, x_mask, edge_index, edge_attr, batch, W_enc, b_enc, W_self, W_nei,
           b_mp, W_dec, b_dec):
    del edge_attr, batch
    x1 = x[:, :_DH]
    x2 = x_mask[:, :_DH]
    w1 = W_enc[:_DH]
    w2 = W_enc[_DH:]
    wsn = jnp.concatenate([W_self, W_nei], axis=1)
    be = b_enc.reshape(1, _DH)
    bm = b_mp.reshape(1, _DH)
    wd = jnp.pad(W_dec, ((0, 0), (0, 8 - W_dec.shape[1])))
    bd = jnp.pad(b_dec, (0, 8 - b_dec.shape[0])).reshape(1, 8)

    pad = _EPAD - _E
    src3 = jnp.concatenate([edge_index[0], jnp.zeros((pad,), jnp.int32)]
                           ).reshape(_NW, _KC, _C)
    dst3 = jnp.concatenate([edge_index[1], jnp.full((pad,), _N, jnp.int32)]
                           ).reshape(_NW, _KC, _C)
    zeros = jnp.zeros((_NPAD, _DH), jnp.float32)

    hw, hn = _tc_pre(x1, x2, w1, w2, be, wsn, bm)
    for _ in range(3):
        parts = _sc_segsum(hn, src3, dst3, zeros)
        hw, hn = _tc_mid(hw, parts, wsn, bm)
    parts = _sc_segsum(hn, src3, dst3, zeros)
    out = _tc_fin(hw, parts, wd, bd)
    return out[:, :3]
